# baseline (speedup 1.0000x reference)
import dataclasses
import functools

import jax
import jax.numpy as jnp
from jax import lax
from jax.experimental import pallas as pl
from jax.experimental.pallas import tpu as pltpu
from jax.experimental.pallas import tpu_sc as plsc

NC = 2
NS = 16
L = 16
NW = NC * NS
B = 16384
BPW = B // NW
CHUNK = 128
NCHUNK = BPW // CHUNK


def kernel(user, item, users_emb, items_emb, W, b):
    user2 = user.reshape(NW * NCHUNK, CHUNK)
    item2 = item.reshape(NW * NCHUNK, CHUNK)
    w16 = W.reshape(L)
    bv = jnp.broadcast_to(b, (L,))

    mesh = plsc.VectorSubcoreMesh(core_axis_name="c", subcore_axis_name="s")
    cp = pltpu.CompilerParams(
        needs_layout_passes=False, use_tc_tiling_on_sc=False)

    @functools.partial(
        pl.kernel,
        out_type=jax.ShapeDtypeStruct((B,), jnp.float32),
        mesh=mesh,
        compiler_params=cp,
        scratch_types=[
            pltpu.VMEM((NCHUNK, CHUNK), jnp.int32),
            pltpu.VMEM((NCHUNK, CHUNK), jnp.int32),
            pltpu.VMEM((BPW, L), jnp.float32),
            pltpu.VMEM((BPW, L), jnp.float32),
            pltpu.VMEM((L,), jnp.float32),
            pltpu.VMEM((L,), jnp.float32),
            pltpu.VMEM((BPW,), jnp.float32),
            pltpu.SemaphoreType.DMA,
            pltpu.SemaphoreType.DMA,
        ],
    )
    def sc_kernel(user_hbm, item_hbm, uemb_hbm, iemb_hbm, w_hbm, bv_hbm,
                  out_hbm, idxu_v, idxi_v, eu_v, ei_v, w_v, bv_v, out_v,
                  sem_idx, sem_g):
        wid = lax.axis_index("s") * NC + lax.axis_index("c")
        row0 = wid * NCHUNK

        pltpu.sync_copy(w_hbm, w_v)
        pltpu.sync_copy(bv_hbm, bv_v)

        cu = pltpu.async_copy(user_hbm.at[pl.ds(row0, NCHUNK)], idxu_v, sem_idx)
        ci = pltpu.async_copy(item_hbm.at[pl.ds(row0, NCHUNK)], idxi_v, sem_idx)
        cu.wait()
        ci.wait()

        gathers = []
        for j in range(NCHUNK):
            gathers.append(pltpu.async_copy(
                uemb_hbm.at[idxu_v.at[j]],
                eu_v.at[pl.ds(j * CHUNK, CHUNK)], sem_g))
            gathers.append(pltpu.async_copy(
                iemb_hbm.at[idxi_v.at[j]],
                ei_v.at[pl.ds(j * CHUNK, CHUNK)], sem_g))
        for g in gathers:
            g.wait()

        wreg = w_v[...]
        breg = bv_v[...]
        lanes = lax.iota(jnp.int32, L)

        @pl.loop(0, BPW // L)
        def _(jb):
            r0 = jb * L
            acc = jnp.zeros((L,), jnp.float32)
            for i in range(L):
                p = eu_v[r0 + i, :] * ei_v[r0 + i, :] * wreg
                s = jnp.sum(p)
                acc = jnp.where(lanes == i, s, acc)
            out_v[pl.ds(r0, L)] = acc + breg

        pltpu.sync_copy(out_v, out_hbm.at[pl.ds(wid * BPW, BPW)])

    return sc_kernel(user2, item2, users_emb, items_emb, w16, bv)
